# baseline (speedup 1.0000x reference)
import functools

import jax
import jax.numpy as jnp
from jax import lax
from jax.experimental import pallas as pl
from jax.experimental.pallas import tpu as pltpu
from jax.experimental.pallas import tpu_sc as plsc

N = 10000
E = 320000
D = 128

NC = 2
NS = 16
NW = NC * NS
CHUNK = 80
NCHUNKS = E // CHUNK
PC = NCHUNKS // NW
NSEG = 5
SEGC = PC // NSEG
RB = 640
IQF = RB // CHUNK
IQL = (N - 15 * RB) // CHUNK
DPAD = 10240
DSL = DPAD // NS

_mesh = plsc.VectorSubcoreMesh(core_axis_name="c", subcore_axis_name="s")
_sc_params = pltpu.CompilerParams(needs_layout_passes=False)


@functools.partial(
    pl.kernel,
    out_type=[jax.ShapeDtypeStruct((DPAD,), jnp.float32)] * 4,
    mesh=_mesh,
    compiler_params=_sc_params,
    scratch_types=[
        pltpu.VMEM((SEGC, CHUNK), jnp.int32),
        pltpu.VMEM((SEGC, CHUNK), jnp.float32),
        pltpu.VMEM((DSL,), jnp.float32),
        pltpu.VMEM_SHARED((DPAD,), jnp.float32),
        pltpu.VMEM_SHARED((DPAD,), jnp.float32),
    ],
)
def _deg_kernel(col1, ew1, col2, ew2, o1a, o1b, o2a, o2b,
                cola, ewa, zbuf, deg1_sh, deg2_sh):
    cid = lax.axis_index("c")
    sid = lax.axis_index("s")
    wid = sid * NC + cid

    def zero16(k, _):
        zbuf[pl.ds(k * 16, 16)] = jnp.zeros((16,), jnp.float32)
        return 0

    lax.fori_loop(0, DSL // 16, zero16, 0)
    sl = pl.ds(sid * DSL, DSL)
    pltpu.sync_copy(zbuf, deg1_sh.at[sl])
    pltpu.sync_copy(zbuf, deg2_sh.at[sl])
    plsc.subcore_barrier()

    for colc, ewc, deg_sh in ((col1, ew1, deg1_sh), (col2, ew2, deg2_sh)):
        def seg_body(sg, _):
            pltpu.sync_copy(colc.at[wid, sg], cola)
            pltpu.sync_copy(ewc.at[wid, sg], ewa)

            def scat(i, _):
                pltpu.sync_copy(ewa.at[i], deg_sh.at[cola.at[i]], add=True)
                return 0

            lax.fori_loop(0, SEGC, scat, 0)
            return 0

        lax.fori_loop(0, NSEG, seg_body, 0)

    plsc.subcore_barrier()
    for deg_sh, oa, ob in ((deg1_sh, o1a, o1b), (deg2_sh, o2a, o2b)):
        pltpu.sync_copy(deg_sh.at[sl], zbuf)

        @pl.when(cid == 0)
        def _():
            pltpu.sync_copy(zbuf, oa.at[sl])

        @pl.when(cid == 1)
        def _():
            pltpu.sync_copy(zbuf, ob.at[sl])


NPAIR = (PC - 1) // 2


@functools.partial(
    pl.kernel,
    out_type=jax.ShapeDtypeStruct((2, NC, N, D), jnp.float32),
    mesh=_mesh,
    compiler_params=_sc_params,
    scratch_types=[
        pltpu.VMEM((N,), jnp.float32),
        pltpu.VMEM((3, CHUNK), jnp.int32),
        pltpu.VMEM((3, CHUNK), jnp.int32),
        pltpu.VMEM((3, CHUNK), jnp.int32),
        pltpu.VMEM((CHUNK, D), jnp.float32),
        pltpu.VMEM((CHUNK, D), jnp.float32),
        pltpu.VMEM((CHUNK, D), jnp.float32),
        pltpu.VMEM_SHARED((N, D), jnp.float32),
        pltpu.SemaphoreType.DMA,
        pltpu.SemaphoreType.DMA,
        pltpu.SemaphoreType.DMA,
        pltpu.SemaphoreType.DMA,
        pltpu.SemaphoreType.DMA,
        pltpu.SemaphoreType.DMA,
    ],
)
def _agg_kernel(h1, dinv1, ed1, h2, dinv2, ed2, out,
                dinv_v, eb0, eb1, eb2, rows0, rows1, rows2,
                acc_sh, gs0, gs1, gs2, ss0, ss1, ss2):
    cid = lax.axis_index("c")
    sid = lax.axis_index("s")
    wid = sid * NC + cid
    base0 = sid * RB
    nq = jnp.where(sid < NS - 1, IQF, IQL)

    for g, (h, dinv, ed) in enumerate(((h1, dinv1, ed1), (h2, dinv2, ed2))):
        pltpu.sync_copy(dinv, dinv_v)

        @pl.when(cid == 0)
        def _():
            def init_chunk(q, _):
                base = base0 + q * CHUNK
                pltpu.sync_copy(h.at[pl.ds(base, CHUNK)], rows0)

                def init_g16(r, _):
                    dv16 = dinv_v[pl.ds(base + r * 16, 16)]
                    s16 = dv16 * dv16
                    for el in range(16):
                        s = s16[el]
                        for k in range(D // 16):
                            ksl = pl.ds(k * 16, 16)
                            rows0[r * 16 + el, ksl] = rows0[r * 16 + el, ksl] * s
                    return 0

                lax.fori_loop(0, CHUNK // 16, init_g16, 0)
                pltpu.sync_copy(rows0, acc_sh.at[pl.ds(base, CHUNK)])
                return 0

            lax.fori_loop(0, nq, init_chunk, 0)

        @pl.when(cid != 0)
        def _():
            def zero_row(r, _):
                for k in range(D // 16):
                    rows0[r, pl.ds(k * 16, 16)] = jnp.zeros((16,), jnp.float32)
                return 0

            lax.fori_loop(0, CHUNK, zero_row, 0)

            def zero_chunk(q, _):
                pltpu.sync_copy(rows0, acc_sh.at[pl.ds(base0 + q * CHUNK, CHUNK)])
                return 0

            lax.fori_loop(0, nq, zero_chunk, 0)

        plsc.subcore_barrier()

        ebs = (eb0, eb1, eb2)
        rows = (rows0, rows1, rows2)
        gss = (gs0, gs1, gs2)
        sss = (ss0, ss1, ss2)

        def load_idx(i, eb):
            pltpu.sync_copy(ed.at[wid, i // SEGC, i % SEGC], eb)

        def start_gather(b, eb):
            pltpu.async_copy(h.at[eb.at[0]], rows[b], gss[b])

        def wait_gather(b, eb):
            pltpu.make_async_copy(h.at[eb.at[0]], rows[b], gss[b]).wait()

        def scale(b):
            eb, rv = ebs[b], rows[b]

            for k in range(CHUNK // 16):
                ksl = pl.ds(k * 16, 16)
                r16 = eb[0, ksl]
                c16 = eb[1, ksl]
                ew16 = plsc.bitcast(eb[2, ksl], jnp.float32)
                w16 = (plsc.load_gather(dinv_v, [r16]) * ew16
                       * plsc.load_gather(dinv_v, [c16]))
                for el in range(16):
                    s = w16[el]
                    for kk in range(D // 16):
                        kksl = pl.ds(kk * 16, 16)
                        rv[k * 16 + el, kksl] = rv[k * 16 + el, kksl] * s

        def start_scatter(b):
            pltpu.async_copy(rows[b], acc_sh.at[ebs[b].at[1]], sss[b], add=True)

        def wait_scatter(b):
            pltpu.make_async_copy(
                rows[b], acc_sh.at[ebs[b].at[1]], sss[b]).wait()

        def chunk_step(i, b, guard, prefetch):
            if guard:
                @pl.when(i >= 2)
                def _():
                    wait_scatter((b + 1) % 3)
            else:
                wait_scatter((b + 1) % 3)
            if prefetch:
                load_idx(i + 1, ebs[(b + 1) % 3])
                start_gather((b + 1) % 3, ebs[(b + 1) % 3])
            wait_gather(b, ebs[b])
            scale(b)
            start_scatter(b)

        load_idx(0, eb0)
        start_gather(0, eb0)

        def triple_body(t, _):
            i = 3 * t
            chunk_step(i, 0, True, True)
            chunk_step(i + 1, 1, True, True)
            chunk_step(i + 2, 2, False, True)
            return 0

        lax.fori_loop(0, (PC - 2) // 3, triple_body, 0)
        chunk_step(PC - 2, 0, False, True)
        chunk_step(PC - 1, 1, False, False)
        wait_scatter(0)
        wait_scatter(1)
        plsc.subcore_barrier()

        def copy_chunk(q, _):
            base = base0 + q * CHUNK
            pltpu.sync_copy(acc_sh.at[pl.ds(base, CHUNK)],
                            out.at[g, cid, pl.ds(base, CHUNK)])
            return 0

        lax.fori_loop(0, nq, copy_chunk, 0)


_ROWS = 1000
_GRID = N // _ROWS


def _mm_body(x1_ref, w1_ref, x2_ref, w2_ref, h1_ref, h2_ref):
    h1_ref[...] = jnp.dot(x1_ref[...], w1_ref[...],
                          preferred_element_type=jnp.float32)
    h2_ref[...] = jnp.dot(x2_ref[...], w2_ref[...],
                          preferred_element_type=jnp.float32)


def _mm_call(x1, w1, x2, w2):
    mat_spec = pl.BlockSpec((_ROWS, D), lambda i: (i, 0))
    w_spec = pl.BlockSpec((D, D), lambda i: (0, 0))
    return pl.pallas_call(
        _mm_body,
        grid=(_GRID,),
        in_specs=[mat_spec, w_spec, mat_spec, w_spec],
        out_specs=[mat_spec, mat_spec],
        out_shape=[
            jax.ShapeDtypeStruct((N, D), jnp.float32),
            jax.ShapeDtypeStruct((N, D), jnp.float32),
        ],
    )(x1, w1, x2, w2)


def _dinv_body(d1a_ref, d1b_ref, d2a_ref, d2b_ref, v1_ref, v2_ref):
    for da, db, v in ((d1a_ref, d1b_ref, v1_ref), (d2a_ref, d2b_ref, v2_ref)):
        deg = da[0, 0, :] + db[0, 0, :] + 1.0
        v[0, 0, :] = jnp.where(
            deg > 0, lax.rsqrt(jnp.maximum(deg, 1e-12)), 0.0)


def _dinv_call(d1a, d1b, d2a, d2b):
    d_spec = pl.BlockSpec((1, 1, _ROWS), lambda i: (i, 0, 0))
    return pl.pallas_call(
        _dinv_body,
        grid=(_GRID,),
        in_specs=[d_spec] * 4,
        out_specs=[d_spec, d_spec],
        out_shape=[
            jax.ShapeDtypeStruct((_GRID, 1, _ROWS), jnp.float32),
            jax.ShapeDtypeStruct((_GRID, 1, _ROWS), jnp.float32),
        ],
    )(d1a, d1b, d2a, d2b)


def _out_body(p1a_ref, p1b_ref, b1_ref, a1_ref, p2a_ref, p2b_ref, b2_ref,
              a2_ref, o_ref):
    y1 = p1a_ref[...] + p1b_ref[...] + b1_ref[...]
    y1 = jnp.where(y1 > 0, y1, y1 * a1_ref[...])
    y2 = p2a_ref[...] + p2b_ref[...] + b2_ref[...]
    y2 = jnp.where(y2 > 0, y2, y2 * a2_ref[...])
    o_ref[...] = (y1 + y2) * 0.5


def _out_call(p1a, p1b, b1, a1, p2a, p2b, b2, a2):
    mat_spec = pl.BlockSpec((_ROWS, D), lambda i: (i, 0))
    vec_spec = pl.BlockSpec((1, D), lambda i: (0, 0))
    return pl.pallas_call(
        _out_body,
        grid=(_GRID,),
        in_specs=[mat_spec, mat_spec, vec_spec, vec_spec,
                  mat_spec, mat_spec, vec_spec, vec_spec],
        out_specs=mat_spec,
        out_shape=jax.ShapeDtypeStruct((N, D), jnp.float32),
    )(p1a, p1b, b1, a1, p2a, p2b, b2, a2)


@jax.jit
def kernel(x1, edge_index1, edge_weight1, x2, edge_index2, edge_weight2,
           W1, b1, W2, b2, a1, a2):
    eshape = (NW, NSEG, SEGC, CHUNK)
    col1 = edge_index1[1].reshape(eshape)
    ew1 = edge_weight1.reshape(eshape)
    col2 = edge_index2[1].reshape(eshape)
    ew2 = edge_weight2.reshape(eshape)
    ed1 = jnp.stack([edge_index1[0].reshape(eshape), col1,
                     lax.bitcast_convert_type(ew1, jnp.int32)], axis=3)
    ed2 = jnp.stack([edge_index2[0].reshape(eshape), col2,
                     lax.bitcast_convert_type(ew2, jnp.int32)], axis=3)

    o1a, o1b, o2a, o2b = _deg_kernel(col1, ew1, col2, ew2)
    d1a = o1a[:N].reshape(_GRID, 1, _ROWS)
    d1b = o1b[:N].reshape(_GRID, 1, _ROWS)
    d2a = o2a[:N].reshape(_GRID, 1, _ROWS)
    d2b = o2b[:N].reshape(_GRID, 1, _ROWS)

    h1, h2 = _mm_call(x1, W1, x2, W2)
    v1, v2 = _dinv_call(d1a, d1b, d2a, d2b)
    dinv1 = v1.reshape(N)
    dinv2 = v2.reshape(N)

    parts = _agg_kernel(h1, dinv1, ed1, h2, dinv2, ed2)

    return _out_call(parts[0, 0], parts[0, 1], b1.reshape(1, D),
                     a1.reshape(1, D), parts[1, 0], parts[1, 1],
                     b2.reshape(1, D), a2.reshape(1, D))


# speedup vs baseline: 1.2914x; 1.2914x over previous
import functools

import jax
import jax.numpy as jnp
from jax import lax
from jax.experimental import pallas as pl
from jax.experimental.pallas import tpu as pltpu
from jax.experimental.pallas import tpu_sc as plsc

N = 10000
E = 320000
D = 128

NC = 2
NS = 16
NW = NC * NS
CHUNK = 80
NCHUNKS = E // CHUNK
PC = NCHUNKS // NW
NSEG = 5
SEGC = PC // NSEG
RB = 640
IQF = RB // CHUNK
IQL = (N - 15 * RB) // CHUNK
DPAD = 10240
DSL = DPAD // NS

_mesh = plsc.VectorSubcoreMesh(core_axis_name="c", subcore_axis_name="s")
_sc_params = pltpu.CompilerParams(needs_layout_passes=False)


@functools.partial(
    pl.kernel,
    out_type=[jax.ShapeDtypeStruct((DPAD,), jnp.float32)] * 4,
    mesh=_mesh,
    compiler_params=_sc_params,
    scratch_types=[
        pltpu.VMEM((SEGC, CHUNK), jnp.int32),
        pltpu.VMEM((SEGC, CHUNK), jnp.float32),
        pltpu.VMEM((DSL,), jnp.float32),
        pltpu.VMEM_SHARED((DPAD,), jnp.float32),
        pltpu.VMEM_SHARED((DPAD,), jnp.float32),
    ],
)
def _deg_kernel(col1, ew1, col2, ew2, o1a, o1b, o2a, o2b,
                cola, ewa, zbuf, deg1_sh, deg2_sh):
    cid = lax.axis_index("c")
    sid = lax.axis_index("s")
    wid = sid * NC + cid

    def zero16(k, _):
        zbuf[pl.ds(k * 16, 16)] = jnp.zeros((16,), jnp.float32)
        return 0

    lax.fori_loop(0, DSL // 16, zero16, 0)
    sl = pl.ds(sid * DSL, DSL)
    pltpu.sync_copy(zbuf, deg1_sh.at[sl])
    pltpu.sync_copy(zbuf, deg2_sh.at[sl])
    plsc.subcore_barrier()

    for colc, ewc, deg_sh in ((col1, ew1, deg1_sh), (col2, ew2, deg2_sh)):
        def seg_body(sg, _):
            pltpu.sync_copy(colc.at[wid, sg], cola)
            pltpu.sync_copy(ewc.at[wid, sg], ewa)

            def scat(i, _):
                pltpu.sync_copy(ewa.at[i], deg_sh.at[cola.at[i]], add=True)
                return 0

            lax.fori_loop(0, SEGC, scat, 0)
            return 0

        lax.fori_loop(0, NSEG, seg_body, 0)

    plsc.subcore_barrier()
    for deg_sh, oa, ob in ((deg1_sh, o1a, o1b), (deg2_sh, o2a, o2b)):
        pltpu.sync_copy(deg_sh.at[sl], zbuf)

        @pl.when(cid == 0)
        def _():
            pltpu.sync_copy(zbuf, oa.at[sl])

        @pl.when(cid == 1)
        def _():
            pltpu.sync_copy(zbuf, ob.at[sl])


NPAIR = (PC - 1) // 2


@functools.partial(
    pl.kernel,
    out_type=jax.ShapeDtypeStruct((2, NC, N, D), jnp.float32),
    mesh=_mesh,
    compiler_params=_sc_params,
    scratch_types=[
        pltpu.VMEM((N,), jnp.float32),
        pltpu.VMEM((3, CHUNK), jnp.int32),
        pltpu.VMEM((3, CHUNK), jnp.int32),
        pltpu.VMEM((3, CHUNK), jnp.int32),
        pltpu.VMEM((CHUNK, D), jnp.float32),
        pltpu.VMEM((CHUNK, D), jnp.float32),
        pltpu.VMEM((CHUNK, D), jnp.float32),
        pltpu.VMEM_SHARED((N, D), jnp.float32),
        pltpu.SemaphoreType.DMA,
        pltpu.SemaphoreType.DMA,
        pltpu.SemaphoreType.DMA,
        pltpu.SemaphoreType.DMA,
        pltpu.SemaphoreType.DMA,
        pltpu.SemaphoreType.DMA,
    ],
)
def _agg_kernel(h1, dinv1, ed1, h2, dinv2, ed2, out,
                dinv_v, eb0, eb1, eb2, rows0, rows1, rows2,
                acc_sh, gs0, gs1, gs2, ss0, ss1, ss2):
    cid = lax.axis_index("c")
    sid = lax.axis_index("s")
    wid = sid * NC + cid
    base0 = sid * RB
    nq = jnp.where(sid < NS - 1, IQF, IQL)

    for g, (h, dinv, ed) in enumerate(((h1, dinv1, ed1), (h2, dinv2, ed2))):
        pltpu.sync_copy(dinv, dinv_v)

        @pl.when(cid == 0)
        def _():
            def init_chunk(q, _):
                base = base0 + q * CHUNK
                pltpu.sync_copy(h.at[pl.ds(base, CHUNK)], rows0)

                def init_g16(r, _):
                    dv16 = dinv_v[pl.ds(base + r * 16, 16)]
                    s16 = dv16 * dv16
                    for el in range(16):
                        s = s16[el]
                        for k in range(D // 16):
                            ksl = pl.ds(k * 16, 16)
                            rows0[r * 16 + el, ksl] = rows0[r * 16 + el, ksl] * s
                    return 0

                lax.fori_loop(0, CHUNK // 16, init_g16, 0)
                pltpu.sync_copy(rows0, acc_sh.at[pl.ds(base, CHUNK)])
                return 0

            lax.fori_loop(0, nq, init_chunk, 0)

        @pl.when(cid != 0)
        def _():
            def zero_row(r, _):
                for k in range(D // 16):
                    rows0[r, pl.ds(k * 16, 16)] = jnp.zeros((16,), jnp.float32)
                return 0

            lax.fori_loop(0, CHUNK, zero_row, 0)

            def zero_chunk(q, _):
                pltpu.sync_copy(rows0, acc_sh.at[pl.ds(base0 + q * CHUNK, CHUNK)])
                return 0

            lax.fori_loop(0, nq, zero_chunk, 0)

        plsc.subcore_barrier()

        ebs = (eb0, eb1, eb2)
        rows = (rows0, rows1, rows2)
        gss = (gs0, gs1, gs2)
        sss = (ss0, ss1, ss2)

        def load_idx(i, eb):
            pltpu.sync_copy(ed.at[wid, i // SEGC, i % SEGC], eb)

        def start_gather(b, eb):
            pltpu.async_copy(h.at[eb.at[0]], rows[b], gss[b])

        def wait_gather(b, eb):
            pltpu.make_async_copy(h.at[eb.at[0]], rows[b], gss[b]).wait()

        def scale(b):
            eb, rv = ebs[b], rows[b]

            def scale_g16(k, _):
                ksl = pl.ds(k * 16, 16)
                r16 = eb[0, ksl]
                c16 = eb[1, ksl]
                ew16 = plsc.bitcast(eb[2, ksl], jnp.float32)
                w16 = (plsc.load_gather(dinv_v, [r16]) * ew16
                       * plsc.load_gather(dinv_v, [c16]))
                for el in range(16):
                    s = w16[el]
                    for kk in range(D // 16):
                        kksl = pl.ds(kk * 16, 16)
                        rv[k * 16 + el, kksl] = rv[k * 16 + el, kksl] * s
                return 0

            lax.fori_loop(0, CHUNK // 16, scale_g16, 0)

        def start_scatter(b):
            pltpu.async_copy(rows[b], acc_sh.at[ebs[b].at[1]], sss[b], add=True)

        def wait_scatter(b):
            pltpu.make_async_copy(
                rows[b], acc_sh.at[ebs[b].at[1]], sss[b]).wait()

        def chunk_step(i, b, guard, prefetch):
            if guard:
                @pl.when(i >= 2)
                def _():
                    wait_scatter((b + 1) % 3)
            else:
                wait_scatter((b + 1) % 3)
            if prefetch:
                load_idx(i + 1, ebs[(b + 1) % 3])
                start_gather((b + 1) % 3, ebs[(b + 1) % 3])
            wait_gather(b, ebs[b])
            scale(b)
            start_scatter(b)

        load_idx(0, eb0)
        start_gather(0, eb0)

        def triple_body(t, _):
            i = 3 * t
            chunk_step(i, 0, True, True)
            chunk_step(i + 1, 1, True, True)
            chunk_step(i + 2, 2, False, True)
            return 0

        lax.fori_loop(0, (PC - 2) // 3, triple_body, 0)
        chunk_step(PC - 2, 0, False, True)
        chunk_step(PC - 1, 1, False, False)
        wait_scatter(0)
        wait_scatter(1)
        plsc.subcore_barrier()

        def copy_chunk(q, _):
            base = base0 + q * CHUNK
            pltpu.sync_copy(acc_sh.at[pl.ds(base, CHUNK)],
                            out.at[g, cid, pl.ds(base, CHUNK)])
            return 0

        lax.fori_loop(0, nq, copy_chunk, 0)


_ROWS = 1000
_GRID = N // _ROWS


def _mm_body(x1_ref, w1_ref, x2_ref, w2_ref, h1_ref, h2_ref):
    h1_ref[...] = jnp.dot(x1_ref[...], w1_ref[...],
                          preferred_element_type=jnp.float32)
    h2_ref[...] = jnp.dot(x2_ref[...], w2_ref[...],
                          preferred_element_type=jnp.float32)


def _mm_call(x1, w1, x2, w2):
    mat_spec = pl.BlockSpec((_ROWS, D), lambda i: (i, 0))
    w_spec = pl.BlockSpec((D, D), lambda i: (0, 0))
    return pl.pallas_call(
        _mm_body,
        grid=(_GRID,),
        in_specs=[mat_spec, w_spec, mat_spec, w_spec],
        out_specs=[mat_spec, mat_spec],
        out_shape=[
            jax.ShapeDtypeStruct((N, D), jnp.float32),
            jax.ShapeDtypeStruct((N, D), jnp.float32),
        ],
    )(x1, w1, x2, w2)


def _dinv_body(d1a_ref, d1b_ref, d2a_ref, d2b_ref, v1_ref, v2_ref):
    for da, db, v in ((d1a_ref, d1b_ref, v1_ref), (d2a_ref, d2b_ref, v2_ref)):
        deg = da[0, 0, :] + db[0, 0, :] + 1.0
        v[0, 0, :] = jnp.where(
            deg > 0, lax.rsqrt(jnp.maximum(deg, 1e-12)), 0.0)


def _dinv_call(d1a, d1b, d2a, d2b):
    d_spec = pl.BlockSpec((1, 1, _ROWS), lambda i: (i, 0, 0))
    return pl.pallas_call(
        _dinv_body,
        grid=(_GRID,),
        in_specs=[d_spec] * 4,
        out_specs=[d_spec, d_spec],
        out_shape=[
            jax.ShapeDtypeStruct((_GRID, 1, _ROWS), jnp.float32),
            jax.ShapeDtypeStruct((_GRID, 1, _ROWS), jnp.float32),
        ],
    )(d1a, d1b, d2a, d2b)


def _out_body(p1a_ref, p1b_ref, b1_ref, a1_ref, p2a_ref, p2b_ref, b2_ref,
              a2_ref, o_ref):
    y1 = p1a_ref[...] + p1b_ref[...] + b1_ref[...]
    y1 = jnp.where(y1 > 0, y1, y1 * a1_ref[...])
    y2 = p2a_ref[...] + p2b_ref[...] + b2_ref[...]
    y2 = jnp.where(y2 > 0, y2, y2 * a2_ref[...])
    o_ref[...] = (y1 + y2) * 0.5


def _out_call(p1a, p1b, b1, a1, p2a, p2b, b2, a2):
    mat_spec = pl.BlockSpec((_ROWS, D), lambda i: (i, 0))
    vec_spec = pl.BlockSpec((1, D), lambda i: (0, 0))
    return pl.pallas_call(
        _out_body,
        grid=(_GRID,),
        in_specs=[mat_spec, mat_spec, vec_spec, vec_spec,
                  mat_spec, mat_spec, vec_spec, vec_spec],
        out_specs=mat_spec,
        out_shape=jax.ShapeDtypeStruct((N, D), jnp.float32),
    )(p1a, p1b, b1, a1, p2a, p2b, b2, a2)


@jax.jit
def kernel(x1, edge_index1, edge_weight1, x2, edge_index2, edge_weight2,
           W1, b1, W2, b2, a1, a2):
    eshape = (NW, NSEG, SEGC, CHUNK)
    col1 = edge_index1[1].reshape(eshape)
    ew1 = edge_weight1.reshape(eshape)
    col2 = edge_index2[1].reshape(eshape)
    ew2 = edge_weight2.reshape(eshape)
    ed1 = jnp.stack([edge_index1[0].reshape(eshape), col1,
                     lax.bitcast_convert_type(ew1, jnp.int32)], axis=3)
    ed2 = jnp.stack([edge_index2[0].reshape(eshape), col2,
                     lax.bitcast_convert_type(ew2, jnp.int32)], axis=3)

    o1a, o1b, o2a, o2b = _deg_kernel(col1, ew1, col2, ew2)
    d1a = o1a[:N].reshape(_GRID, 1, _ROWS)
    d1b = o1b[:N].reshape(_GRID, 1, _ROWS)
    d2a = o2a[:N].reshape(_GRID, 1, _ROWS)
    d2b = o2b[:N].reshape(_GRID, 1, _ROWS)

    h1, h2 = _mm_call(x1, W1, x2, W2)
    v1, v2 = _dinv_call(d1a, d1b, d2a, d2b)
    dinv1 = v1.reshape(N)
    dinv2 = v2.reshape(N)

    parts = _agg_kernel(h1, dinv1, ed1, h2, dinv2, ed2)

    return _out_call(parts[0, 0], parts[0, 1], b1.reshape(1, D),
                     a1.reshape(1, D), parts[1, 0], parts[1, 1],
                     b2.reshape(1, D), a2.reshape(1, D))


# speedup vs baseline: 1.4850x; 1.1499x over previous
import functools

import jax
import jax.numpy as jnp
from jax import lax
from jax.experimental import pallas as pl
from jax.experimental.pallas import tpu as pltpu
from jax.experimental.pallas import tpu_sc as plsc

N = 10000
E = 320000
D = 128

NC = 2
NS = 16
NW = NC * NS
CHUNK = 80
NCHUNKS = E // CHUNK
PC = NCHUNKS // NW
NSEG = 5
SEGC = PC // NSEG
RB = 640
IQF = RB // CHUNK
IQL = (N - 15 * RB) // CHUNK
DPAD = 10240
DSL = DPAD // NS

_mesh = plsc.VectorSubcoreMesh(core_axis_name="c", subcore_axis_name="s")
_sc_params = pltpu.CompilerParams(needs_layout_passes=False)


@functools.partial(
    pl.kernel,
    out_type=[jax.ShapeDtypeStruct((DPAD,), jnp.float32)] * 4,
    mesh=_mesh,
    compiler_params=_sc_params,
    scratch_types=[
        pltpu.VMEM((SEGC, CHUNK), jnp.int32),
        pltpu.VMEM((SEGC, CHUNK), jnp.float32),
        pltpu.VMEM((DSL,), jnp.float32),
        pltpu.VMEM_SHARED((DPAD,), jnp.float32),
        pltpu.VMEM_SHARED((DPAD,), jnp.float32),
    ],
)
def _deg_kernel(col1, ew1, col2, ew2, o1a, o1b, o2a, o2b,
                cola, ewa, zbuf, deg1_sh, deg2_sh):
    cid = lax.axis_index("c")
    sid = lax.axis_index("s")
    wid = sid * NC + cid

    def zero16(k, _):
        zbuf[pl.ds(k * 16, 16)] = jnp.zeros((16,), jnp.float32)
        return 0

    lax.fori_loop(0, DSL // 16, zero16, 0)
    sl = pl.ds(sid * DSL, DSL)
    pltpu.sync_copy(zbuf, deg1_sh.at[sl])
    pltpu.sync_copy(zbuf, deg2_sh.at[sl])
    plsc.subcore_barrier()

    for colc, ewc, deg_sh in ((col1, ew1, deg1_sh), (col2, ew2, deg2_sh)):
        def seg_body(sg, _):
            pltpu.sync_copy(colc.at[wid, sg], cola)
            pltpu.sync_copy(ewc.at[wid, sg], ewa)

            def scat(i, _):
                pltpu.sync_copy(ewa.at[i], deg_sh.at[cola.at[i]], add=True)
                return 0

            lax.fori_loop(0, SEGC, scat, 0)
            return 0

        lax.fori_loop(0, NSEG, seg_body, 0)

    plsc.subcore_barrier()
    for deg_sh, oa, ob in ((deg1_sh, o1a, o1b), (deg2_sh, o2a, o2b)):
        pltpu.sync_copy(deg_sh.at[sl], zbuf)

        @pl.when(cid == 0)
        def _():
            pltpu.sync_copy(zbuf, oa.at[sl])

        @pl.when(cid == 1)
        def _():
            pltpu.sync_copy(zbuf, ob.at[sl])


NPAIR = (PC - 1) // 2


@functools.partial(
    pl.kernel,
    out_type=jax.ShapeDtypeStruct((2, NC, N, D), jnp.float32),
    mesh=_mesh,
    compiler_params=_sc_params,
    scratch_types=[
        pltpu.VMEM((N,), jnp.float32),
        pltpu.VMEM((3, CHUNK), jnp.int32),
        pltpu.VMEM((3, CHUNK), jnp.int32),
        pltpu.VMEM((3, CHUNK), jnp.int32),
        pltpu.VMEM((CHUNK, D), jnp.float32),
        pltpu.VMEM((CHUNK, D), jnp.float32),
        pltpu.VMEM((CHUNK, D), jnp.float32),
        pltpu.VMEM((CHUNK,), jnp.int32),
        pltpu.VMEM((CHUNK,), jnp.int32),
        pltpu.VMEM((CHUNK,), jnp.int32),
        pltpu.VMEM_SHARED((N, D), jnp.float32),
        pltpu.SemaphoreType.DMA,
        pltpu.SemaphoreType.DMA,
        pltpu.SemaphoreType.DMA,
        pltpu.SemaphoreType.DMA,
        pltpu.SemaphoreType.DMA,
        pltpu.SemaphoreType.DMA,
        pltpu.SemaphoreType.DMA,
        pltpu.SemaphoreType.DMA,
        pltpu.SemaphoreType.DMA,
    ],
)
def _agg_kernel(h1, dinv1, ed1, h2, dinv2, ed2, out,
                dinv_v, eb0, eb1, eb2, rows0, rows1, rows2, ci0, ci1, ci2,
                acc_sh, gs0, gs1, gs2, ss0, ss1, ss2, is0, is1, is2):
    cid = lax.axis_index("c")
    sid = lax.axis_index("s")
    wid = sid * NC + cid
    base0 = sid * RB
    nq = jnp.where(sid < NS - 1, IQF, IQL)

    for g, (h, dinv, ed) in enumerate(((h1, dinv1, ed1), (h2, dinv2, ed2))):
        pltpu.sync_copy(dinv, dinv_v)

        @pl.when(cid == g % NC)
        def _():
            def init_chunk(q, _):
                base = base0 + q * CHUNK
                pltpu.sync_copy(h.at[pl.ds(base, CHUNK)], rows0)

                def init_g16(r, _):
                    dv16 = dinv_v[pl.ds(base + r * 16, 16)]
                    s16 = dv16 * dv16
                    for el in range(16):
                        s = s16[el]
                        for k in range(D // 16):
                            ksl = pl.ds(k * 16, 16)
                            rows0[r * 16 + el, ksl] = rows0[r * 16 + el, ksl] * s
                    return 0

                lax.fori_loop(0, CHUNK // 16, init_g16, 0)
                pltpu.sync_copy(rows0, acc_sh.at[pl.ds(base, CHUNK)])
                return 0

            lax.fori_loop(0, nq, init_chunk, 0)

        @pl.when(cid != g % NC)
        def _():
            def zero_row(r, _):
                for k in range(D // 16):
                    rows0[r, pl.ds(k * 16, 16)] = jnp.zeros((16,), jnp.float32)
                return 0

            lax.fori_loop(0, CHUNK, zero_row, 0)

            def zero_chunk(q, _):
                pltpu.sync_copy(rows0, acc_sh.at[pl.ds(base0 + q * CHUNK, CHUNK)])
                return 0

            lax.fori_loop(0, nq, zero_chunk, 0)

        plsc.subcore_barrier()

        ebs = (eb0, eb1, eb2)
        rows = (rows0, rows1, rows2)
        cis = (ci0, ci1, ci2)
        gss = (gs0, gs1, gs2)
        sss = (ss0, ss1, ss2)
        iss = (is0, is1, is2)

        def start_idx(i, b):
            i = jnp.minimum(i, PC - 1)
            pltpu.async_copy(ed.at[wid, i // SEGC, i % SEGC], ebs[b], iss[b])

        def wait_idx(b):
            pltpu.make_async_copy(ed.at[wid, 0, 0], ebs[b], iss[b]).wait()

        def start_gather(b):
            pltpu.async_copy(h.at[ebs[b].at[0]], rows[b], gss[b])

        def wait_gather(b):
            pltpu.make_async_copy(h.at[ebs[b].at[0]], rows[b], gss[b]).wait()

        def scale(b):
            eb, rv, ci = ebs[b], rows[b], cis[b]

            def scale_g16(k, _):
                ksl = pl.ds(k * 16, 16)
                r16 = eb[0, ksl]
                c16 = eb[1, ksl]
                ci[ksl] = c16
                ew16 = plsc.bitcast(eb[2, ksl], jnp.float32)
                w16 = (plsc.load_gather(dinv_v, [r16]) * ew16
                       * plsc.load_gather(dinv_v, [c16]))
                for el in range(16):
                    s = w16[el]
                    for kk in range(D // 16):
                        kksl = pl.ds(kk * 16, 16)
                        rv[k * 16 + el, kksl] = rv[k * 16 + el, kksl] * s
                return 0

            lax.fori_loop(0, CHUNK // 16, scale_g16, 0)

        def start_scatter(b):
            pltpu.async_copy(rows[b], acc_sh.at[cis[b]], sss[b], add=True)

        def wait_scatter(b):
            pltpu.make_async_copy(rows[b], acc_sh.at[cis[b]], sss[b]).wait()

        def chunk_step(i, b, guard):
            if guard:
                @pl.when(i >= 2)
                def _():
                    wait_scatter((b + 1) % 3)
            else:
                wait_scatter((b + 1) % 3)
            start_idx(i + 2, (b + 2) % 3)
            wait_idx((b + 1) % 3)
            start_gather((b + 1) % 3)
            wait_gather(b)
            scale(b)
            start_scatter(b)

        start_idx(0, 0)
        start_idx(1, 1)
        wait_idx(0)
        start_gather(0)

        def triple_body(t, _):
            i = 3 * t
            chunk_step(i, 0, True)
            chunk_step(i + 1, 1, True)
            chunk_step(i + 2, 2, False)
            return 0

        lax.fori_loop(0, (PC - 2) // 3, triple_body, 0)
        chunk_step(PC - 2, 0, False)
        wait_scatter(2)
        wait_idx(2)
        wait_gather(1)
        scale(1)
        start_scatter(1)
        wait_scatter(0)
        wait_scatter(1)
        plsc.subcore_barrier()

        def copy_chunk(q, _):
            base = base0 + q * CHUNK
            pltpu.sync_copy(acc_sh.at[pl.ds(base, CHUNK)],
                            out.at[g, cid, pl.ds(base, CHUNK)])
            return 0

        lax.fori_loop(0, nq, copy_chunk, 0)


_ROWS = 1000
_GRID = N // _ROWS


def _mm_body(x1_ref, w1_ref, x2_ref, w2_ref, h1_ref, h2_ref):
    h1_ref[...] = jnp.dot(x1_ref[...], w1_ref[...],
                          preferred_element_type=jnp.float32)
    h2_ref[...] = jnp.dot(x2_ref[...], w2_ref[...],
                          preferred_element_type=jnp.float32)


def _mm_call(x1, w1, x2, w2):
    mat_spec = pl.BlockSpec((_ROWS, D), lambda i: (i, 0))
    w_spec = pl.BlockSpec((D, D), lambda i: (0, 0))
    return pl.pallas_call(
        _mm_body,
        grid=(_GRID,),
        in_specs=[mat_spec, w_spec, mat_spec, w_spec],
        out_specs=[mat_spec, mat_spec],
        out_shape=[
            jax.ShapeDtypeStruct((N, D), jnp.float32),
            jax.ShapeDtypeStruct((N, D), jnp.float32),
        ],
    )(x1, w1, x2, w2)


def _dinv_body(d1a_ref, d1b_ref, d2a_ref, d2b_ref, v1_ref, v2_ref):
    for da, db, v in ((d1a_ref, d1b_ref, v1_ref), (d2a_ref, d2b_ref, v2_ref)):
        deg = da[0, 0, :] + db[0, 0, :] + 1.0
        v[0, 0, :] = jnp.where(
            deg > 0, lax.rsqrt(jnp.maximum(deg, 1e-12)), 0.0)


def _dinv_call(d1a, d1b, d2a, d2b):
    d_spec = pl.BlockSpec((1, 1, _ROWS), lambda i: (i, 0, 0))
    return pl.pallas_call(
        _dinv_body,
        grid=(_GRID,),
        in_specs=[d_spec] * 4,
        out_specs=[d_spec, d_spec],
        out_shape=[
            jax.ShapeDtypeStruct((_GRID, 1, _ROWS), jnp.float32),
            jax.ShapeDtypeStruct((_GRID, 1, _ROWS), jnp.float32),
        ],
    )(d1a, d1b, d2a, d2b)


def _out_body(p1a_ref, p1b_ref, b1_ref, a1_ref, p2a_ref, p2b_ref, b2_ref,
              a2_ref, o_ref):
    y1 = p1a_ref[...] + p1b_ref[...] + b1_ref[...]
    y1 = jnp.where(y1 > 0, y1, y1 * a1_ref[...])
    y2 = p2a_ref[...] + p2b_ref[...] + b2_ref[...]
    y2 = jnp.where(y2 > 0, y2, y2 * a2_ref[...])
    o_ref[...] = (y1 + y2) * 0.5


def _out_call(p1a, p1b, b1, a1, p2a, p2b, b2, a2):
    mat_spec = pl.BlockSpec((_ROWS, D), lambda i: (i, 0))
    vec_spec = pl.BlockSpec((1, D), lambda i: (0, 0))
    return pl.pallas_call(
        _out_body,
        grid=(_GRID,),
        in_specs=[mat_spec, mat_spec, vec_spec, vec_spec,
                  mat_spec, mat_spec, vec_spec, vec_spec],
        out_specs=mat_spec,
        out_shape=jax.ShapeDtypeStruct((N, D), jnp.float32),
    )(p1a, p1b, b1, a1, p2a, p2b, b2, a2)


@jax.jit
def kernel(x1, edge_index1, edge_weight1, x2, edge_index2, edge_weight2,
           W1, b1, W2, b2, a1, a2):
    eshape = (NW, NSEG, SEGC, CHUNK)
    col1 = edge_index1[1].reshape(eshape)
    ew1 = edge_weight1.reshape(eshape)
    col2 = edge_index2[1].reshape(eshape)
    ew2 = edge_weight2.reshape(eshape)
    ed1 = jnp.stack([edge_index1[0].reshape(eshape), col1,
                     lax.bitcast_convert_type(ew1, jnp.int32)], axis=3)
    ed2 = jnp.stack([edge_index2[0].reshape(eshape), col2,
                     lax.bitcast_convert_type(ew2, jnp.int32)], axis=3)

    o1a, o1b, o2a, o2b = _deg_kernel(col1, ew1, col2, ew2)
    d1a = o1a[:N].reshape(_GRID, 1, _ROWS)
    d1b = o1b[:N].reshape(_GRID, 1, _ROWS)
    d2a = o2a[:N].reshape(_GRID, 1, _ROWS)
    d2b = o2b[:N].reshape(_GRID, 1, _ROWS)

    h1, h2 = _mm_call(x1, W1, x2, W2)
    v1, v2 = _dinv_call(d1a, d1b, d2a, d2b)
    dinv1 = v1.reshape(N)
    dinv2 = v2.reshape(N)

    parts = _agg_kernel(h1, dinv1, ed1, h2, dinv2, ed2)

    return _out_call(parts[0, 0], parts[0, 1], b1.reshape(1, D),
                     a1.reshape(1, D), parts[1, 0], parts[1, 1],
                     b2.reshape(1, D), a2.reshape(1, D))


# speedup vs baseline: 1.4880x; 1.0020x over previous
import functools

import jax
import jax.numpy as jnp
from jax import lax
from jax.experimental import pallas as pl
from jax.experimental.pallas import tpu as pltpu
from jax.experimental.pallas import tpu_sc as plsc

N = 10000
E = 320000
D = 128

NC = 2
NS = 16
NW = NC * NS
CHUNK = 80
NCHUNKS = E // CHUNK
PC = NCHUNKS // NW
NSEG = 5
SEGC = PC // NSEG
RB = 640
IQF = RB // CHUNK
IQL = (N - 15 * RB) // CHUNK
DPAD = 10240
DSL = DPAD // NS

_mesh = plsc.VectorSubcoreMesh(core_axis_name="c", subcore_axis_name="s")
_sc_params = pltpu.CompilerParams(needs_layout_passes=False)


@functools.partial(
    pl.kernel,
    out_type=[jax.ShapeDtypeStruct((DPAD,), jnp.float32)] * 4,
    mesh=_mesh,
    compiler_params=_sc_params,
    scratch_types=[
        pltpu.VMEM((SEGC, CHUNK), jnp.int32),
        pltpu.VMEM((SEGC, CHUNK), jnp.float32),
        pltpu.VMEM((DSL,), jnp.float32),
        pltpu.VMEM_SHARED((DPAD,), jnp.float32),
        pltpu.VMEM_SHARED((DPAD,), jnp.float32),
        pltpu.SemaphoreType.DMA,
    ],
)
def _deg_kernel(col1, ew1, col2, ew2, o1a, o1b, o2a, o2b,
                cola, ewa, zbuf, deg1_sh, deg2_sh, ssem):
    cid = lax.axis_index("c")
    sid = lax.axis_index("s")
    wid = sid * NC + cid

    def zero16(k, _):
        zbuf[pl.ds(k * 16, 16)] = jnp.zeros((16,), jnp.float32)
        return 0

    lax.fori_loop(0, DSL // 16, zero16, 0)
    sl = pl.ds(sid * DSL, DSL)
    pltpu.sync_copy(zbuf, deg1_sh.at[sl])
    pltpu.sync_copy(zbuf, deg2_sh.at[sl])
    plsc.subcore_barrier()

    for colc, ewc, deg_sh in ((col1, ew1, deg1_sh), (col2, ew2, deg2_sh)):
        def seg_body(sg, _):
            pltpu.sync_copy(colc.at[wid, sg], cola)
            pltpu.sync_copy(ewc.at[wid, sg], ewa)

            def scat(i, _):
                pltpu.async_copy(ewa.at[i], deg_sh.at[cola.at[i]], ssem,
                                 add=True)
                return 0

            lax.fori_loop(0, SEGC, scat, 0)

            def drain(i, _):
                pltpu.make_async_copy(
                    ewa.at[i], deg_sh.at[cola.at[i]], ssem).wait()
                return 0

            lax.fori_loop(0, SEGC, drain, 0)
            return 0

        lax.fori_loop(0, NSEG, seg_body, 0)

    plsc.subcore_barrier()
    for deg_sh, oa, ob in ((deg1_sh, o1a, o1b), (deg2_sh, o2a, o2b)):
        pltpu.sync_copy(deg_sh.at[sl], zbuf)

        @pl.when(cid == 0)
        def _():
            pltpu.sync_copy(zbuf, oa.at[sl])

        @pl.when(cid == 1)
        def _():
            pltpu.sync_copy(zbuf, ob.at[sl])


NPAIR = (PC - 1) // 2


@functools.partial(
    pl.kernel,
    out_type=jax.ShapeDtypeStruct((2, NC, N, D), jnp.float32),
    mesh=_mesh,
    compiler_params=_sc_params,
    scratch_types=[
        pltpu.VMEM((N,), jnp.float32),
        pltpu.VMEM((3, CHUNK), jnp.int32),
        pltpu.VMEM((3, CHUNK), jnp.int32),
        pltpu.VMEM((3, CHUNK), jnp.int32),
        pltpu.VMEM((CHUNK, D), jnp.float32),
        pltpu.VMEM((CHUNK, D), jnp.float32),
        pltpu.VMEM((CHUNK, D), jnp.float32),
        pltpu.VMEM((CHUNK,), jnp.int32),
        pltpu.VMEM((CHUNK,), jnp.int32),
        pltpu.VMEM((CHUNK,), jnp.int32),
        pltpu.VMEM_SHARED((N, D), jnp.float32),
        pltpu.SemaphoreType.DMA,
        pltpu.SemaphoreType.DMA,
        pltpu.SemaphoreType.DMA,
        pltpu.SemaphoreType.DMA,
        pltpu.SemaphoreType.DMA,
        pltpu.SemaphoreType.DMA,
        pltpu.SemaphoreType.DMA,
        pltpu.SemaphoreType.DMA,
        pltpu.SemaphoreType.DMA,
    ],
)
def _agg_kernel(h1, dinv1, ed1, h2, dinv2, ed2, out,
                dinv_v, eb0, eb1, eb2, rows0, rows1, rows2, ci0, ci1, ci2,
                acc_sh, gs0, gs1, gs2, ss0, ss1, ss2, is0, is1, is2):
    cid = lax.axis_index("c")
    sid = lax.axis_index("s")
    wid = sid * NC + cid
    base0 = sid * RB
    nq = jnp.where(sid < NS - 1, IQF, IQL)

    for g, (h, dinv, ed) in enumerate(((h1, dinv1, ed1), (h2, dinv2, ed2))):
        pltpu.sync_copy(dinv, dinv_v)

        @pl.when(cid == g % NC)
        def _():
            def init_chunk(q, _):
                base = base0 + q * CHUNK
                pltpu.sync_copy(h.at[pl.ds(base, CHUNK)], rows0)

                def init_g16(r, _):
                    dv16 = dinv_v[pl.ds(base + r * 16, 16)]
                    s16 = dv16 * dv16
                    for el in range(16):
                        s = s16[el]
                        for k in range(D // 16):
                            ksl = pl.ds(k * 16, 16)
                            rows0[r * 16 + el, ksl] = rows0[r * 16 + el, ksl] * s
                    return 0

                lax.fori_loop(0, CHUNK // 16, init_g16, 0)
                pltpu.sync_copy(rows0, acc_sh.at[pl.ds(base, CHUNK)])
                return 0

            lax.fori_loop(0, nq, init_chunk, 0)

        @pl.when(cid != g % NC)
        def _():
            def zero_row(r, _):
                for k in range(D // 16):
                    rows0[r, pl.ds(k * 16, 16)] = jnp.zeros((16,), jnp.float32)
                return 0

            lax.fori_loop(0, CHUNK, zero_row, 0)

            def zero_chunk(q, _):
                pltpu.sync_copy(rows0, acc_sh.at[pl.ds(base0 + q * CHUNK, CHUNK)])
                return 0

            lax.fori_loop(0, nq, zero_chunk, 0)

        plsc.subcore_barrier()

        ebs = (eb0, eb1, eb2)
        rows = (rows0, rows1, rows2)
        cis = (ci0, ci1, ci2)
        gss = (gs0, gs1, gs2)
        sss = (ss0, ss1, ss2)
        iss = (is0, is1, is2)

        def start_idx(i, b):
            i = jnp.minimum(i, PC - 1)
            pltpu.async_copy(ed.at[wid, i // SEGC, i % SEGC], ebs[b], iss[b])

        def wait_idx(b):
            pltpu.make_async_copy(ed.at[wid, 0, 0], ebs[b], iss[b]).wait()

        def start_gather(b):
            pltpu.async_copy(h.at[ebs[b].at[0]], rows[b], gss[b])

        def wait_gather(b):
            pltpu.make_async_copy(h.at[ebs[b].at[0]], rows[b], gss[b]).wait()

        def scale(b):
            eb, rv, ci = ebs[b], rows[b], cis[b]

            def scale_g16(k, _):
                ksl = pl.ds(k * 16, 16)
                r16 = eb[0, ksl]
                c16 = eb[1, ksl]
                ci[ksl] = c16
                ew16 = plsc.bitcast(eb[2, ksl], jnp.float32)
                w16 = (plsc.load_gather(dinv_v, [r16]) * ew16
                       * plsc.load_gather(dinv_v, [c16]))
                for el in range(16):
                    s = w16[el]
                    for kk in range(D // 16):
                        kksl = pl.ds(kk * 16, 16)
                        rv[k * 16 + el, kksl] = rv[k * 16 + el, kksl] * s
                return 0

            lax.fori_loop(0, CHUNK // 16, scale_g16, 0)

        def start_scatter(b):
            pltpu.async_copy(rows[b], acc_sh.at[cis[b]], sss[b], add=True)

        def wait_scatter(b):
            pltpu.make_async_copy(rows[b], acc_sh.at[cis[b]], sss[b]).wait()

        def chunk_step(i, b, guard):
            if guard:
                @pl.when(i >= 2)
                def _():
                    wait_scatter((b + 1) % 3)
            else:
                wait_scatter((b + 1) % 3)
            start_idx(i + 2, (b + 2) % 3)
            wait_idx((b + 1) % 3)
            start_gather((b + 1) % 3)
            wait_gather(b)
            scale(b)
            start_scatter(b)

        start_idx(0, 0)
        start_idx(1, 1)
        wait_idx(0)
        start_gather(0)

        def triple_body(t, _):
            i = 3 * t
            chunk_step(i, 0, True)
            chunk_step(i + 1, 1, True)
            chunk_step(i + 2, 2, False)
            return 0

        lax.fori_loop(0, (PC - 2) // 3, triple_body, 0)
        chunk_step(PC - 2, 0, False)
        wait_scatter(2)
        wait_idx(2)
        wait_gather(1)
        scale(1)
        start_scatter(1)
        wait_scatter(0)
        wait_scatter(1)
        plsc.subcore_barrier()

        def copy_chunk(q, _):
            base = base0 + q * CHUNK
            pltpu.sync_copy(acc_sh.at[pl.ds(base, CHUNK)],
                            out.at[g, cid, pl.ds(base, CHUNK)])
            return 0

        lax.fori_loop(0, nq, copy_chunk, 0)


_ROWS = 1000
_GRID = N // _ROWS


def _mm_body(x1_ref, w1_ref, x2_ref, w2_ref, h1_ref, h2_ref):
    h1_ref[...] = jnp.dot(x1_ref[...], w1_ref[...],
                          preferred_element_type=jnp.float32)
    h2_ref[...] = jnp.dot(x2_ref[...], w2_ref[...],
                          preferred_element_type=jnp.float32)


def _mm_call(x1, w1, x2, w2):
    mat_spec = pl.BlockSpec((_ROWS, D), lambda i: (i, 0))
    w_spec = pl.BlockSpec((D, D), lambda i: (0, 0))
    return pl.pallas_call(
        _mm_body,
        grid=(_GRID,),
        in_specs=[mat_spec, w_spec, mat_spec, w_spec],
        out_specs=[mat_spec, mat_spec],
        out_shape=[
            jax.ShapeDtypeStruct((N, D), jnp.float32),
            jax.ShapeDtypeStruct((N, D), jnp.float32),
        ],
    )(x1, w1, x2, w2)


def _dinv_body(d1a_ref, d1b_ref, d2a_ref, d2b_ref, v1_ref, v2_ref):
    for da, db, v in ((d1a_ref, d1b_ref, v1_ref), (d2a_ref, d2b_ref, v2_ref)):
        deg = da[0, 0, :] + db[0, 0, :] + 1.0
        v[0, 0, :] = jnp.where(
            deg > 0, lax.rsqrt(jnp.maximum(deg, 1e-12)), 0.0)


def _dinv_call(d1a, d1b, d2a, d2b):
    d_spec = pl.BlockSpec((1, 1, _ROWS), lambda i: (i, 0, 0))
    return pl.pallas_call(
        _dinv_body,
        grid=(_GRID,),
        in_specs=[d_spec] * 4,
        out_specs=[d_spec, d_spec],
        out_shape=[
            jax.ShapeDtypeStruct((_GRID, 1, _ROWS), jnp.float32),
            jax.ShapeDtypeStruct((_GRID, 1, _ROWS), jnp.float32),
        ],
    )(d1a, d1b, d2a, d2b)


def _out_body(p1a_ref, p1b_ref, b1_ref, a1_ref, p2a_ref, p2b_ref, b2_ref,
              a2_ref, o_ref):
    y1 = p1a_ref[...] + p1b_ref[...] + b1_ref[...]
    y1 = jnp.where(y1 > 0, y1, y1 * a1_ref[...])
    y2 = p2a_ref[...] + p2b_ref[...] + b2_ref[...]
    y2 = jnp.where(y2 > 0, y2, y2 * a2_ref[...])
    o_ref[...] = (y1 + y2) * 0.5


def _out_call(p1a, p1b, b1, a1, p2a, p2b, b2, a2):
    mat_spec = pl.BlockSpec((_ROWS, D), lambda i: (i, 0))
    vec_spec = pl.BlockSpec((1, D), lambda i: (0, 0))
    return pl.pallas_call(
        _out_body,
        grid=(_GRID,),
        in_specs=[mat_spec, mat_spec, vec_spec, vec_spec,
                  mat_spec, mat_spec, vec_spec, vec_spec],
        out_specs=mat_spec,
        out_shape=jax.ShapeDtypeStruct((N, D), jnp.float32),
    )(p1a, p1b, b1, a1, p2a, p2b, b2, a2)


@jax.jit
def kernel(x1, edge_index1, edge_weight1, x2, edge_index2, edge_weight2,
           W1, b1, W2, b2, a1, a2):
    eshape = (NW, NSEG, SEGC, CHUNK)
    col1 = edge_index1[1].reshape(eshape)
    ew1 = edge_weight1.reshape(eshape)
    col2 = edge_index2[1].reshape(eshape)
    ew2 = edge_weight2.reshape(eshape)
    ed1 = jnp.stack([edge_index1[0].reshape(eshape), col1,
                     lax.bitcast_convert_type(ew1, jnp.int32)], axis=3)
    ed2 = jnp.stack([edge_index2[0].reshape(eshape), col2,
                     lax.bitcast_convert_type(ew2, jnp.int32)], axis=3)

    o1a, o1b, o2a, o2b = _deg_kernel(col1, ew1, col2, ew2)
    d1a = o1a[:N].reshape(_GRID, 1, _ROWS)
    d1b = o1b[:N].reshape(_GRID, 1, _ROWS)
    d2a = o2a[:N].reshape(_GRID, 1, _ROWS)
    d2b = o2b[:N].reshape(_GRID, 1, _ROWS)

    h1, h2 = _mm_call(x1, W1, x2, W2)
    v1, v2 = _dinv_call(d1a, d1b, d2a, d2b)
    dinv1 = v1.reshape(N)
    dinv2 = v2.reshape(N)

    parts = _agg_kernel(h1, dinv1, ed1, h2, dinv2, ed2)

    return _out_call(parts[0, 0], parts[0, 1], b1.reshape(1, D),
                     a1.reshape(1, D), parts[1, 0], parts[1, 1],
                     b2.reshape(1, D), a2.reshape(1, D))


# speedup vs baseline: 1.5743x; 1.0580x over previous
import functools

import jax
import jax.numpy as jnp
from jax import lax
from jax.experimental import pallas as pl
from jax.experimental.pallas import tpu as pltpu
from jax.experimental.pallas import tpu_sc as plsc

N = 10000
E = 320000
D = 128

NC = 2
NS = 16
NW = NC * NS
CHUNK = 80
NCHUNKS = E // CHUNK
PC = NCHUNKS // NW
NSEG = 5
SEGC = PC // NSEG
RB = 640
IQF = RB // CHUNK
IQL = (N - 15 * RB) // CHUNK
DPAD = 10240
DSL = DPAD // NS

_mesh = plsc.VectorSubcoreMesh(core_axis_name="c", subcore_axis_name="s")
_sc_params = pltpu.CompilerParams(needs_layout_passes=False)


@functools.partial(
    pl.kernel,
    out_type=[jax.ShapeDtypeStruct((DPAD,), jnp.float32)] * 4,
    mesh=_mesh,
    compiler_params=_sc_params,
    scratch_types=[
        pltpu.VMEM((SEGC, CHUNK), jnp.int32),
        pltpu.VMEM((SEGC, CHUNK), jnp.float32),
        pltpu.VMEM((DSL,), jnp.float32),
        pltpu.VMEM_SHARED((DPAD,), jnp.float32),
        pltpu.VMEM_SHARED((DPAD,), jnp.float32),
        pltpu.SemaphoreType.DMA,
    ],
)
def _deg_kernel(col1, ew1, col2, ew2, o1a, o1b, o2a, o2b,
                cola, ewa, zbuf, deg1_sh, deg2_sh, ssem):
    cid = lax.axis_index("c")
    sid = lax.axis_index("s")
    wid = sid * NC + cid

    def zero16(k, _):
        zbuf[pl.ds(k * 16, 16)] = jnp.zeros((16,), jnp.float32)
        return 0

    lax.fori_loop(0, DSL // 16, zero16, 0)
    sl = pl.ds(sid * DSL, DSL)
    pltpu.sync_copy(zbuf, deg1_sh.at[sl])
    pltpu.sync_copy(zbuf, deg2_sh.at[sl])
    plsc.subcore_barrier()

    for colc, ewc, deg_sh in ((col1, ew1, deg1_sh), (col2, ew2, deg2_sh)):
        def seg_body(sg, _):
            pltpu.sync_copy(colc.at[wid, sg], cola)
            pltpu.sync_copy(ewc.at[wid, sg], ewa)

            def scat(i, _):
                pltpu.async_copy(ewa.at[i], deg_sh.at[cola.at[i]], ssem,
                                 add=True)
                return 0

            lax.fori_loop(0, SEGC, scat, 0)

            def drain(i, _):
                pltpu.make_async_copy(
                    ewa.at[i], deg_sh.at[cola.at[i]], ssem).wait()
                return 0

            lax.fori_loop(0, SEGC, drain, 0)
            return 0

        lax.fori_loop(0, NSEG, seg_body, 0)

    plsc.subcore_barrier()
    for deg_sh, oa, ob in ((deg1_sh, o1a, o1b), (deg2_sh, o2a, o2b)):
        pltpu.sync_copy(deg_sh.at[sl], zbuf)

        @pl.when(cid == 0)
        def _():
            pltpu.sync_copy(zbuf, oa.at[sl])

        @pl.when(cid == 1)
        def _():
            pltpu.sync_copy(zbuf, ob.at[sl])


NPAIR = (PC - 1) // 2


@functools.partial(
    pl.kernel,
    out_type=jax.ShapeDtypeStruct((2, NC, N, D), jnp.float32),
    mesh=_mesh,
    compiler_params=_sc_params,
    scratch_types=[
        pltpu.VMEM((N,), jnp.float32),
        pltpu.VMEM((3, CHUNK), jnp.int32),
        pltpu.VMEM((3, CHUNK), jnp.int32),
        pltpu.VMEM((3, CHUNK), jnp.int32),
        pltpu.VMEM((CHUNK, D), jnp.float32),
        pltpu.VMEM((CHUNK, D), jnp.float32),
        pltpu.VMEM((CHUNK, D), jnp.float32),
        pltpu.VMEM((CHUNK,), jnp.int32),
        pltpu.VMEM((CHUNK,), jnp.int32),
        pltpu.VMEM((CHUNK,), jnp.int32),
        pltpu.VMEM_SHARED((N, D), jnp.float32),
        pltpu.SemaphoreType.DMA,
        pltpu.SemaphoreType.DMA,
        pltpu.SemaphoreType.DMA,
        pltpu.SemaphoreType.DMA,
        pltpu.SemaphoreType.DMA,
        pltpu.SemaphoreType.DMA,
        pltpu.SemaphoreType.DMA,
        pltpu.SemaphoreType.DMA,
        pltpu.SemaphoreType.DMA,
    ],
)
def _agg_kernel(h1, dinv1, ed1, h2, dinv2, ed2, out,
                dinv_v, eb0, eb1, eb2, rows0, rows1, rows2, ci0, ci1, ci2,
                acc_sh, gs0, gs1, gs2, ss0, ss1, ss2, is0, is1, is2):
    cid = lax.axis_index("c")
    sid = lax.axis_index("s")
    wid = sid * NC + cid
    base0 = sid * RB
    nq = jnp.where(sid < NS - 1, IQF, IQL)

    for g, (h, dinv, ed) in enumerate(((h1, dinv1, ed1), (h2, dinv2, ed2))):
        pltpu.sync_copy(dinv, dinv_v)

        @pl.when(cid == g % NC)
        def _():
            def init_chunk(q, _):
                base = base0 + q * CHUNK
                pltpu.sync_copy(h.at[pl.ds(base, CHUNK)], rows0)

                def init_g16(r, _):
                    dv16 = dinv_v[pl.ds(base + r * 16, 16)]
                    s16 = dv16 * dv16
                    for el in range(16):
                        s = s16[el]
                        for k in range(D // 16):
                            ksl = pl.ds(k * 16, 16)
                            rows0[r * 16 + el, ksl] = rows0[r * 16 + el, ksl] * s
                    return 0

                lax.fori_loop(0, CHUNK // 16, init_g16, 0)
                pltpu.sync_copy(rows0, acc_sh.at[pl.ds(base, CHUNK)])
                return 0

            lax.fori_loop(0, nq, init_chunk, 0)

        @pl.when(cid != g % NC)
        def _():
            def zero_row(r, _):
                for k in range(D // 16):
                    rows0[r, pl.ds(k * 16, 16)] = jnp.zeros((16,), jnp.float32)
                return 0

            lax.fori_loop(0, CHUNK, zero_row, 0)

            def zero_chunk(q, _):
                pltpu.sync_copy(rows0, acc_sh.at[pl.ds(base0 + q * CHUNK, CHUNK)])
                return 0

            lax.fori_loop(0, nq, zero_chunk, 0)

        plsc.subcore_barrier()

        ebs = (eb0, eb1, eb2)
        rows = (rows0, rows1, rows2)
        cis = (ci0, ci1, ci2)
        gss = (gs0, gs1, gs2)
        sss = (ss0, ss1, ss2)
        iss = (is0, is1, is2)

        def start_idx(i, b):
            i = jnp.minimum(i, PC - 1)
            pltpu.async_copy(ed.at[wid, i // SEGC, i % SEGC], ebs[b], iss[b])

        def wait_idx(b):
            pltpu.make_async_copy(ed.at[wid, 0, 0], ebs[b], iss[b]).wait()

        def start_gather(b):
            pltpu.async_copy(h.at[ebs[b].at[0]], rows[b], gss[b])

        def wait_gather(b):
            pltpu.make_async_copy(h.at[ebs[b].at[0]], rows[b], gss[b]).wait()

        def scale(b):
            eb, rv, ci = ebs[b], rows[b], cis[b]

            def scale_g16(k, _):
                ksl = pl.ds(k * 16, 16)
                r16 = eb[0, ksl]
                c16 = eb[1, ksl]
                ci[ksl] = c16
                ew16 = plsc.bitcast(eb[2, ksl], jnp.float32)
                w16 = (plsc.load_gather(dinv_v, [r16]) * ew16
                       * plsc.load_gather(dinv_v, [c16]))
                for el in range(16):
                    s = w16[el]
                    for kk in range(D // 16):
                        kksl = pl.ds(kk * 16, 16)
                        rv[k * 16 + el, kksl] = rv[k * 16 + el, kksl] * s
                return 0

            lax.fori_loop(0, CHUNK // 16, scale_g16, 0)

        def start_scatter(b):
            pltpu.async_copy(rows[b], acc_sh.at[cis[b]], sss[b], add=True)

        def wait_scatter(b):
            pltpu.make_async_copy(rows[b], acc_sh.at[cis[b]], sss[b]).wait()

        def chunk_step(i, b, guard):
            if guard:
                @pl.when(i >= 2)
                def _():
                    wait_scatter((b + 1) % 3)
            else:
                wait_scatter((b + 1) % 3)
            start_idx(i + 2, (b + 2) % 3)
            wait_idx((b + 1) % 3)
            start_gather((b + 1) % 3)
            wait_gather(b)
            scale(b)
            start_scatter(b)

        start_idx(0, 0)
        start_idx(1, 1)
        wait_idx(0)
        start_gather(0)

        def triple_body(t, _):
            i = 3 * t
            chunk_step(i, 0, True)
            chunk_step(i + 1, 1, True)
            chunk_step(i + 2, 2, False)
            return 0

        lax.fori_loop(0, (PC - 2) // 3, triple_body, 0)
        chunk_step(PC - 2, 0, False)
        wait_scatter(2)
        wait_idx(2)
        wait_gather(1)
        scale(1)
        start_scatter(1)
        wait_scatter(0)
        wait_scatter(1)
        plsc.subcore_barrier()

        def copy_chunk(q, _):
            base = base0 + q * CHUNK
            pltpu.sync_copy(acc_sh.at[pl.ds(base, CHUNK)],
                            out.at[g, cid, pl.ds(base, CHUNK)])
            return 0

        lax.fori_loop(0, nq, copy_chunk, 0)


_ROWS = 1000
_GRID = N // _ROWS


def _dense_body(x1_ref, w1_ref, d1a_ref, d1b_ref, x2_ref, w2_ref, d2a_ref,
                d2b_ref, h1_ref, h2_ref, v1_ref, v2_ref):
    h1_ref[...] = jnp.dot(x1_ref[...], w1_ref[...],
                          preferred_element_type=jnp.float32)
    h2_ref[...] = jnp.dot(x2_ref[...], w2_ref[...],
                          preferred_element_type=jnp.float32)
    for da, db, v in ((d1a_ref, d1b_ref, v1_ref), (d2a_ref, d2b_ref, v2_ref)):
        deg = da[0, 0, :] + db[0, 0, :] + 1.0
        v[0, 0, :] = jnp.where(
            deg > 0, lax.rsqrt(jnp.maximum(deg, 1e-12)), 0.0)


def _dense_call(x1, w1, d1a, d1b, x2, w2, d2a, d2b):
    mat_spec = pl.BlockSpec((_ROWS, D), lambda i: (i, 0))
    w_spec = pl.BlockSpec((D, D), lambda i: (0, 0))
    d_spec = pl.BlockSpec((1, 1, _ROWS), lambda i: (i, 0, 0))
    return pl.pallas_call(
        _dense_body,
        grid=(_GRID,),
        in_specs=[mat_spec, w_spec, d_spec, d_spec,
                  mat_spec, w_spec, d_spec, d_spec],
        out_specs=[mat_spec, mat_spec, d_spec, d_spec],
        out_shape=[
            jax.ShapeDtypeStruct((N, D), jnp.float32),
            jax.ShapeDtypeStruct((N, D), jnp.float32),
            jax.ShapeDtypeStruct((_GRID, 1, _ROWS), jnp.float32),
            jax.ShapeDtypeStruct((_GRID, 1, _ROWS), jnp.float32),
        ],
    )(x1, w1, d1a, d1b, x2, w2, d2a, d2b)


def _out_body(p1a_ref, p1b_ref, b1_ref, a1_ref, p2a_ref, p2b_ref, b2_ref,
              a2_ref, o_ref):
    y1 = p1a_ref[0, 0] + p1b_ref[0, 0] + b1_ref[...]
    y1 = jnp.where(y1 > 0, y1, y1 * a1_ref[...])
    y2 = p2a_ref[0, 0] + p2b_ref[0, 0] + b2_ref[...]
    y2 = jnp.where(y2 > 0, y2, y2 * a2_ref[...])
    o_ref[...] = (y1 + y2) * 0.5


def _out_call(parts, b1, a1, b2, a2):
    mat_spec = pl.BlockSpec((_ROWS, D), lambda i: (i, 0))
    vec_spec = pl.BlockSpec((1, D), lambda i: (0, 0))

    def pspec(g, c):
        return pl.BlockSpec((1, 1, _ROWS, D), lambda i: (g, c, i, 0))

    return pl.pallas_call(
        _out_body,
        grid=(_GRID,),
        in_specs=[pspec(0, 0), pspec(0, 1), vec_spec, vec_spec,
                  pspec(1, 0), pspec(1, 1), vec_spec, vec_spec],
        out_specs=mat_spec,
        out_shape=jax.ShapeDtypeStruct((N, D), jnp.float32),
    )(parts, parts, b1, a1, parts, parts, b2, a2)


@jax.jit
def kernel(x1, edge_index1, edge_weight1, x2, edge_index2, edge_weight2,
           W1, b1, W2, b2, a1, a2):
    eshape = (NW, NSEG, SEGC, CHUNK)
    col1 = edge_index1[1].reshape(eshape)
    ew1 = edge_weight1.reshape(eshape)
    col2 = edge_index2[1].reshape(eshape)
    ew2 = edge_weight2.reshape(eshape)
    ed1 = jnp.stack([edge_index1[0].reshape(eshape), col1,
                     lax.bitcast_convert_type(ew1, jnp.int32)], axis=3)
    ed2 = jnp.stack([edge_index2[0].reshape(eshape), col2,
                     lax.bitcast_convert_type(ew2, jnp.int32)], axis=3)

    o1a, o1b, o2a, o2b = _deg_kernel(col1, ew1, col2, ew2)
    d1a = o1a[:N].reshape(_GRID, 1, _ROWS)
    d1b = o1b[:N].reshape(_GRID, 1, _ROWS)
    d2a = o2a[:N].reshape(_GRID, 1, _ROWS)
    d2b = o2b[:N].reshape(_GRID, 1, _ROWS)

    h1, h2, v1, v2 = _dense_call(x1, W1, d1a, d1b, x2, W2, d2a, d2b)
    dinv1 = v1.reshape(N)
    dinv2 = v2.reshape(N)

    parts = _agg_kernel(h1, dinv1, ed1, h2, dinv2, ed2)

    return _out_call(parts, b1.reshape(1, D), a1.reshape(1, D),
                     b2.reshape(1, D), a2.reshape(1, D))
